# baseline (speedup 1.0000x reference)
import functools

import jax
import jax.numpy as jnp
from jax import lax
from jax.experimental import pallas as pl
from jax.experimental.pallas import tpu as pltpu
from jax.experimental.pallas import tpu_sc as plsc

N_WORKERS = 32
NPW = 320
NPAD = N_WORKERS * NPW
EB = 128
KB = 4
OC = 64
LANES = 16
MASK_HI = -65536


def _make_pack_kernel(N, D):
    CH = 80
    nch_full = NPW // CH
    rows_last = N - NPW * (N_WORKERS - 1)
    nch_last = (rows_last + CH - 1) // CH
    assert rows_last % CH == 0 and N % CH == 0
    mesh = plsc.VectorSubcoreMesh(core_axis_name="c", subcore_axis_name="s")

    @functools.partial(
        pl.kernel,
        mesh=mesh,
        out_type=jax.ShapeDtypeStruct((N, D // 2), jnp.int32),
        scratch_types=[
            pltpu.VMEM((2 * CH, D), jnp.float32),
            pltpu.VMEM((2 * CH, D // 2), jnp.int32),
            pltpu.SemaphoreType.DMA,
            pltpu.SemaphoreType.DMA,
            pltpu.SemaphoreType.DMA,
            pltpu.SemaphoreType.DMA,
        ],
    )
    def pack(x_hbm, out_hbm, buf_in, buf_out, sin0, sin1, sout0, sout1):
        wid = lax.axis_index("s") * 2 + lax.axis_index("c")
        base = wid * NPW
        nch = jnp.where(wid < N_WORKERS - 1, nch_full, nch_last)

        in_slc = (buf_in.at[pl.ds(0, CH)], buf_in.at[pl.ds(CH, CH)])
        out_slc = (buf_out.at[pl.ds(0, CH)], buf_out.at[pl.ds(CH, CH)])
        sins = (sin0, sin1)
        souts = (sout0, sout1)

        def src_at(c):
            return pl.multiple_of(base + c * CH, 8)

        def copy_in(c, k):
            pltpu.async_copy(x_hbm.at[pl.ds(src_at(c), CH)], in_slc[k],
                             sins[k])

        def wait_in(c, k):
            pltpu.make_async_copy(x_hbm.at[pl.ds(src_at(c), CH)], in_slc[k],
                                  sins[k]).wait()

        def copy_out(c, k):
            pltpu.async_copy(out_slc[k], out_hbm.at[pl.ds(src_at(c), CH)],
                             souts[k])

        def wait_out(c, k):
            pltpu.make_async_copy(out_slc[k],
                                  out_hbm.at[pl.ds(src_at(c), CH)],
                                  souts[k]).wait()

        copy_in(0, 0)

        half = jnp.full((LANES,), 0x8000, jnp.int32)
        mask_hi = jnp.full((LANES,), MASK_HI, jnp.int32)
        sh16 = jnp.full((LANES,), 16, jnp.int32)

        @pl.loop(0, nch)
        def chunk_loop(c):
            par = jnp.bitwise_and(c, 1)
            for k in range(2):
                @pl.when(par == k)
                def _():
                    wait_in(c, k)

                    @pl.when(c + 1 < nch)
                    def _():
                        copy_in(c + 1, 1 - k)

                    @pl.when(c >= 2)
                    def _():
                        wait_out(c - 2, k)

                    @pl.loop(0, CH)
                    def row_loop(r):
                        rr = r + k * CH
                        for m in range(D // (2 * LANES)):
                            lo = buf_in[rr, pl.ds(2 * m * LANES, LANES)]
                            hi = buf_in[rr, pl.ds((2 * m + 1) * LANES,
                                                  LANES)]
                            li = lax.bitcast_convert_type(lo, jnp.int32)
                            hi_i = lax.bitcast_convert_type(hi, jnp.int32)
                            pk = jnp.bitwise_or(
                                lax.shift_right_logical(li + half, sh16),
                                jnp.bitwise_and(hi_i + half, mask_hi),
                            )
                            buf_out[rr, pl.ds(m * LANES, LANES)] = pk

                    copy_out(c, k)

        @pl.when(nch >= 2)
        def _():
            for k in range(2):
                @pl.when(jnp.bitwise_and(nch - 2, 1) == k)
                def _():
                    wait_out(nch - 2, k)
        for k in range(2):
            @pl.when(jnp.bitwise_and(nch - 1, 1) == k)
            def _():
                wait_out(nch - 1, k)

    return pack


def _make_agg_kernel(D, E):
    nv = D // LANES
    nw = D // (2 * LANES)
    mesh = plsc.VectorSubcoreMesh(core_axis_name="c", subcore_axis_name="s")

    @functools.partial(
        pl.kernel,
        mesh=mesh,
        out_type=jax.ShapeDtypeStruct((NPAD, D), jnp.float32),
        scratch_types=[
            pltpu.VMEM((NPW + 16,), jnp.int32),
            pltpu.VMEM((KB * EB,), jnp.int32),
            pltpu.VMEM((KB * EB, D // 2), jnp.int32),
            pltpu.VMEM((OC, D), jnp.float32),
        ] + [pltpu.SemaphoreType.DMA] * (2 * KB),
    )
    def agg(x_hbm, ptr_hbm, idx_hbm, out_hbm, ptr_v, idx_v, rows_v, out_v,
            *sems):
        sis = sems[:KB]
        srs = sems[KB:]
        wid = lax.axis_index("s") * 2 + lax.axis_index("c")
        base = wid * NPW
        pltpu.sync_copy(ptr_hbm.at[pl.ds(base, NPW + 16)], ptr_v)

        e0 = ptr_v[pl.ds(0, LANES)][0]
        e0a = e0 - jnp.bitwise_and(e0, 7)
        e0a = pl.multiple_of(e0a, 8)

        idx_slc = tuple(idx_v.at[pl.ds(k * EB, EB)] for k in range(KB))
        row_slc = tuple(rows_v.at[pl.ds(k * EB, EB)] for k in range(KB))

        def idx_copy(b, k):
            pltpu.async_copy(
                idx_hbm.at[pl.ds(e0a + b * EB, EB)], idx_slc[k], sis[k]
            )

        def idx_wait(b, k):
            pltpu.make_async_copy(
                idx_hbm.at[pl.ds(e0a + b * EB, EB)], idx_slc[k], sis[k]
            ).wait()

        def row_gather(k):
            pltpu.async_copy(x_hbm.at[idx_slc[k]], row_slc[k], srs[k])

        def row_wait(k):
            pltpu.make_async_copy(
                x_hbm.at[idx_slc[k]], row_slc[k], srs[k]
            ).wait()

        for k in range(KB):
            idx_copy(k, k)
        idx_wait(0, 0)
        row_gather(0)
        idx_wait(1, 1)
        row_gather(1)

        zero = jnp.zeros((LANES,), jnp.float32)
        mask_hi = jnp.full((LANES,), MASK_HI, jnp.int32)
        sh16 = jnp.full((LANES,), 16, jnp.int32)

        def accum(acc, pos):
            nacc = []
            for j in range(nw):
                v = rows_v[pos, pl.ds(j * LANES, LANES)]
                even = lax.bitcast_convert_type(
                    lax.shift_left(v, sh16), jnp.float32
                )
                odd = lax.bitcast_convert_type(
                    jnp.bitwise_and(v, mask_hi), jnp.float32
                )
                nacc.append(acc[2 * j] + even)
                nacc.append(acc[2 * j + 1] + odd)
            return tuple(nacc)

        def accum2(acc, pos):
            nacc = []
            for j in range(nw):
                v1 = rows_v[pos, pl.ds(j * LANES, LANES)]
                v2 = rows_v[pos + 1, pl.ds(j * LANES, LANES)]
                e1 = lax.bitcast_convert_type(
                    lax.shift_left(v1, sh16), jnp.float32
                )
                o1 = lax.bitcast_convert_type(
                    jnp.bitwise_and(v1, mask_hi), jnp.float32
                )
                e2 = lax.bitcast_convert_type(
                    lax.shift_left(v2, sh16), jnp.float32
                )
                o2 = lax.bitcast_convert_type(
                    jnp.bitwise_and(v2, mask_hi), jnp.float32
                )
                nacc.append(acc[2 * j] + (e1 + e2))
                nacc.append(acc[2 * j + 1] + (o1 + o2))
            return tuple(nacc)

        def node_body(n, loaded):
            pv = ptr_v[pl.ds(n, LANES)]
            s = pv[0]
            t = pv[1]
            b_lo = lax.shift_right_logical(s - e0a, 7)
            b_hi = jnp.where(
                t > s, lax.shift_right_logical(t - 1 - e0a, 7) + 1, b_lo
            )

            @pl.loop(b_lo, b_hi, init_carry=(loaded, (zero,) * nv))
            def batch_loop(b, carry):
                loaded, acc = carry
                par = jnp.bitwise_and(b, KB - 1)

                @pl.when(b != loaded)
                def _():
                    for k in range(KB):
                        @pl.when(par == k)
                        def _():
                            row_wait(k)
                            idx_wait(b + 2, (k + 2) % KB)
                            row_gather((k + 2) % KB)
                            idx_copy(b + 4, k)

                bs = e0a + b * EB
                el = jnp.maximum(s, bs)
                eh = jnp.minimum(t, bs + EB)
                off = par * EB - bs
                cnt = eh - el
                pairs = lax.shift_right_logical(cnt, 1)

                @pl.loop(0, pairs, init_carry=acc)
                def pair_loop(i, acc):
                    return accum2(acc, el + 2 * i + off)

                acc = pair_loop
                tail = accum((zero,) * nv, eh - 1 + off)
                wt = jnp.broadcast_to(
                    jnp.bitwise_and(cnt, 1), (LANES,)
                ).astype(jnp.float32)
                acc = tuple(a + tv * wt for a, tv in zip(acc, tail))
                return (b, acc)

            loaded, acc = batch_loop
            cnt = jnp.broadcast_to(
                jnp.maximum(t - s, 1), (LANES,)
            ).astype(jnp.float32)
            scale = jnp.ones((LANES,), jnp.float32) / cnt
            slot = jnp.bitwise_and(n, OC - 1)
            for k in range(nv):
                out_v[slot, pl.ds(k * LANES, LANES)] = acc[k] * scale

            @pl.when(slot == OC - 1)
            def _():
                dst = pl.multiple_of(base + n - (OC - 1), OC)
                pltpu.sync_copy(out_v, out_hbm.at[pl.ds(dst, OC)])

            return loaded

        loaded = lax.fori_loop(0, NPW, node_body, jnp.int32(-1))

        lpar = jnp.bitwise_and(loaded, KB - 1)
        for k in range(KB):
            @pl.when(lpar == k)
            def _():
                row_wait((k + 1) % KB)
                row_wait((k + 2) % KB)
                idx_wait(loaded + 3, (k + 3) % KB)
                idx_wait(loaded + 4, k)

    return agg


def _mm_kernel(agg_ref, x_ref, wl_ref, wr_ref, b_ref, o_ref):
    a = agg_ref[...].astype(jnp.bfloat16)
    xb = x_ref[...].astype(jnp.bfloat16)
    o_ref[...] = (
        jnp.dot(a, wl_ref[...], preferred_element_type=jnp.float32)
        + jnp.dot(xb, wr_ref[...], preferred_element_type=jnp.float32)
        + b_ref[...]
    )


def kernel(x, ptr, idx, num_node, W_l, b_l, W_r):
    N, D = x.shape
    H = W_l.shape[1]
    E = idx.shape[0]

    ptr = ptr.astype(jnp.int32)
    idx = idx.astype(jnp.int32)
    ptr_pad = jnp.concatenate(
        [ptr, jnp.full((NPAD + 16 - (N + 1),), ptr[-1], jnp.int32)]
    )
    idx_pad = jnp.concatenate([idx, jnp.zeros((6 * EB + 8,), jnp.int32)])

    x_pk = _make_pack_kernel(N, D)(x)

    agg = _make_agg_kernel(D, E)(x_pk, ptr_pad, idx_pad)

    b2 = b_l.reshape(1, H)

    BN = 1000
    out = pl.pallas_call(
        _mm_kernel,
        grid=(N // BN,),
        in_specs=[
            pl.BlockSpec((BN, D), lambda i: (i, 0)),
            pl.BlockSpec((BN, D), lambda i: (i, 0)),
            pl.BlockSpec((D, H), lambda i: (0, 0)),
            pl.BlockSpec((D, H), lambda i: (0, 0)),
            pl.BlockSpec((1, H), lambda i: (0, 0)),
        ],
        out_specs=pl.BlockSpec((BN, H), lambda i: (i, 0)),
        out_shape=jax.ShapeDtypeStruct((N, H), jnp.float32),
    )(agg, x, W_l.astype(jnp.bfloat16), W_r.astype(jnp.bfloat16), b2)

    return out


# speedup vs baseline: 1.0908x; 1.0908x over previous
import functools

import jax
import jax.numpy as jnp
from jax import lax
from jax.experimental import pallas as pl
from jax.experimental.pallas import tpu as pltpu
from jax.experimental.pallas import tpu_sc as plsc

N_WORKERS = 32
NPW = 320
NPAD = N_WORKERS * NPW
EB = 128
KB = 4
OC = 64
LANES = 16
MASK_HI = -65536


def _make_pack_kernel(N, D):
    CH = 80
    nch_full = NPW // CH
    rows_last = N - NPW * (N_WORKERS - 1)
    nch_last = (rows_last + CH - 1) // CH
    assert rows_last % CH == 0 and N % CH == 0
    mesh = plsc.VectorSubcoreMesh(core_axis_name="c", subcore_axis_name="s")

    @functools.partial(
        pl.kernel,
        mesh=mesh,
        out_type=jax.ShapeDtypeStruct((N, D // 2), jnp.int32),
        scratch_types=[
            pltpu.VMEM((2 * CH, D), jnp.float32),
            pltpu.VMEM((2 * CH, D // 2), jnp.int32),
            pltpu.SemaphoreType.DMA,
            pltpu.SemaphoreType.DMA,
            pltpu.SemaphoreType.DMA,
            pltpu.SemaphoreType.DMA,
        ],
    )
    def pack(x_hbm, out_hbm, buf_in, buf_out, sin0, sin1, sout0, sout1):
        wid = lax.axis_index("s") * 2 + lax.axis_index("c")
        base = wid * NPW
        nch = jnp.where(wid < N_WORKERS - 1, nch_full, nch_last)

        in_slc = (buf_in.at[pl.ds(0, CH)], buf_in.at[pl.ds(CH, CH)])
        out_slc = (buf_out.at[pl.ds(0, CH)], buf_out.at[pl.ds(CH, CH)])
        sins = (sin0, sin1)
        souts = (sout0, sout1)

        def src_at(c):
            return pl.multiple_of(base + c * CH, 8)

        def copy_in(c, k):
            pltpu.async_copy(x_hbm.at[pl.ds(src_at(c), CH)], in_slc[k],
                             sins[k])

        def wait_in(c, k):
            pltpu.make_async_copy(x_hbm.at[pl.ds(src_at(c), CH)], in_slc[k],
                                  sins[k]).wait()

        def copy_out(c, k):
            pltpu.async_copy(out_slc[k], out_hbm.at[pl.ds(src_at(c), CH)],
                             souts[k])

        def wait_out(c, k):
            pltpu.make_async_copy(out_slc[k],
                                  out_hbm.at[pl.ds(src_at(c), CH)],
                                  souts[k]).wait()

        copy_in(0, 0)

        half = jnp.full((LANES,), 0x8000, jnp.int32)
        mask_hi = jnp.full((LANES,), MASK_HI, jnp.int32)
        sh16 = jnp.full((LANES,), 16, jnp.int32)

        @pl.loop(0, nch)
        def chunk_loop(c):
            par = jnp.bitwise_and(c, 1)
            for k in range(2):
                @pl.when(par == k)
                def _():
                    wait_in(c, k)

                    @pl.when(c + 1 < nch)
                    def _():
                        copy_in(c + 1, 1 - k)

                    @pl.when(c >= 2)
                    def _():
                        wait_out(c - 2, k)

                    @pl.loop(0, CH)
                    def row_loop(r):
                        rr = r + k * CH
                        for m in range(D // (2 * LANES)):
                            lo = buf_in[rr, pl.ds(2 * m * LANES, LANES)]
                            hi = buf_in[rr, pl.ds((2 * m + 1) * LANES,
                                                  LANES)]
                            li = lax.bitcast_convert_type(lo, jnp.int32)
                            hi_i = lax.bitcast_convert_type(hi, jnp.int32)
                            pk = jnp.bitwise_or(
                                lax.shift_right_logical(li + half, sh16),
                                jnp.bitwise_and(hi_i + half, mask_hi),
                            )
                            buf_out[rr, pl.ds(m * LANES, LANES)] = pk

                    copy_out(c, k)

        @pl.when(nch >= 2)
        def _():
            for k in range(2):
                @pl.when(jnp.bitwise_and(nch - 2, 1) == k)
                def _():
                    wait_out(nch - 2, k)
        for k in range(2):
            @pl.when(jnp.bitwise_and(nch - 1, 1) == k)
            def _():
                wait_out(nch - 1, k)

    return pack


def _make_agg_kernel(D, E):
    nv = D // LANES
    nw = D // (2 * LANES)
    mesh = plsc.VectorSubcoreMesh(core_axis_name="c", subcore_axis_name="s")

    @functools.partial(
        pl.kernel,
        mesh=mesh,
        out_type=jax.ShapeDtypeStruct((NPAD, D), jnp.float32),
        scratch_types=[
            pltpu.VMEM((NPW + 16,), jnp.int32),
            pltpu.VMEM((2 * EB,), jnp.int32),
            pltpu.VMEM((2 * EB, D // 2), jnp.int32),
            pltpu.VMEM((OC, D), jnp.float32),
            pltpu.SemaphoreType.DMA,
            pltpu.SemaphoreType.DMA,
            pltpu.SemaphoreType.DMA,
            pltpu.SemaphoreType.DMA,
        ],
    )
    def agg(x_hbm, ptr_hbm, idx_hbm, out_hbm, ptr_v, idx_v, rows_v, out_v,
            si0, si1, sr0, sr1):
        wid = lax.axis_index("s") * 2 + lax.axis_index("c")
        base = wid * NPW
        pltpu.sync_copy(ptr_hbm.at[pl.ds(base, NPW + 16)], ptr_v)

        e0 = ptr_v[pl.ds(0, LANES)][0]
        e0a = e0 - jnp.bitwise_and(e0, 7)
        e0a = pl.multiple_of(e0a, 8)

        idx_slc = (idx_v.at[pl.ds(0, EB)], idx_v.at[pl.ds(EB, EB)])
        row_slc = (rows_v.at[pl.ds(0, EB)], rows_v.at[pl.ds(EB, EB)])
        sis = (si0, si1)
        srs = (sr0, sr1)

        def idx_copy(b, par):
            pltpu.async_copy(
                idx_hbm.at[pl.ds(e0a + b * EB, EB)], idx_slc[par], sis[par]
            )

        def idx_wait(b, par):
            pltpu.make_async_copy(
                idx_hbm.at[pl.ds(e0a + b * EB, EB)], idx_slc[par], sis[par]
            ).wait()

        def row_gather(par):
            pltpu.async_copy(x_hbm.at[idx_slc[par]], row_slc[par], srs[par])

        def row_wait(par):
            pltpu.make_async_copy(
                x_hbm.at[idx_slc[par]], row_slc[par], srs[par]
            ).wait()

        idx_copy(0, 0)
        idx_copy(1, 1)
        idx_wait(0, 0)
        row_gather(0)

        zero = jnp.zeros((LANES,), jnp.float32)

        def node_body(n, loaded):
            pv = ptr_v[pl.ds(n, LANES)]
            s = pv[0]
            t = pv[1]
            b_lo = lax.shift_right_logical(s - e0a, 7)
            b_hi = jnp.where(
                t > s, lax.shift_right_logical(t - 1 - e0a, 7) + 1, b_lo
            )

            @pl.loop(b_lo, b_hi, init_carry=(loaded, (zero,) * nv))
            def batch_loop(b, carry):
                loaded, acc = carry
                par = jnp.bitwise_and(b, 1)

                @pl.when(b != loaded)
                def _():
                    @pl.when(par == 0)
                    def _():
                        row_wait(0)
                        idx_wait(1, 1)
                        idx_copy(b + 2, 0)
                        row_gather(1)

                    @pl.when(par == 1)
                    def _():
                        row_wait(1)
                        idx_wait(0, 0)
                        idx_copy(b + 2, 1)
                        row_gather(0)

                bs = e0a + b * EB
                el = jnp.maximum(s, bs)
                eh = jnp.minimum(t, bs + EB)
                off = par * EB - bs

                mask_hi = jnp.full((LANES,), -65536, jnp.int32)
                sh16 = jnp.full((LANES,), 16, jnp.int32)

                @pl.loop(el, eh, init_carry=acc)
                def edge_loop(e, acc):
                    pos = e + off
                    nacc = []
                    for j in range(nw):
                        v = rows_v[pos, pl.ds(j * LANES, LANES)]
                        even = lax.bitcast_convert_type(
                            lax.shift_left(v, sh16), jnp.float32
                        )
                        odd = lax.bitcast_convert_type(
                            jnp.bitwise_and(v, mask_hi), jnp.float32
                        )
                        nacc.append(acc[2 * j] + even)
                        nacc.append(acc[2 * j + 1] + odd)
                    return tuple(nacc)

                return (b, edge_loop)

            loaded, acc = batch_loop
            cnt = jnp.broadcast_to(
                jnp.maximum(t - s, 1), (LANES,)
            ).astype(jnp.float32)
            scale = jnp.ones((LANES,), jnp.float32) / cnt
            slot = jnp.bitwise_and(n, OC - 1)
            for k in range(nv):
                out_v[slot, pl.ds(k * LANES, LANES)] = acc[k] * scale

            @pl.when(slot == OC - 1)
            def _():
                dst = pl.multiple_of(base + n - (OC - 1), OC)
                pltpu.sync_copy(out_v, out_hbm.at[pl.ds(dst, OC)])

            return loaded

        loaded = lax.fori_loop(0, NPW, node_body, jnp.int32(-1))

        lpar = jnp.bitwise_and(loaded, 1)

        @pl.when(lpar == 0)
        def _():
            idx_wait(loaded + 2, 0)
            row_wait(1)

        @pl.when(lpar == 1)
        def _():
            idx_wait(loaded + 2, 1)
            row_wait(0)

    return agg


def _mm_kernel(agg_ref, x_ref, wl_ref, wr_ref, b_ref, o_ref):
    a = agg_ref[...].astype(jnp.bfloat16)
    xb = x_ref[...].astype(jnp.bfloat16)
    o_ref[...] = (
        jnp.dot(a, wl_ref[...], preferred_element_type=jnp.float32)
        + jnp.dot(xb, wr_ref[...], preferred_element_type=jnp.float32)
        + b_ref[...]
    )


def kernel(x, ptr, idx, num_node, W_l, b_l, W_r):
    N, D = x.shape
    H = W_l.shape[1]
    E = idx.shape[0]

    ptr = ptr.astype(jnp.int32)
    idx = idx.astype(jnp.int32)
    ptr_pad = jnp.concatenate(
        [ptr, jnp.full((NPAD + 16 - (N + 1),), ptr[-1], jnp.int32)]
    )
    idx_pad = jnp.concatenate([idx, jnp.zeros((6 * EB + 8,), jnp.int32)])

    x_pk = _make_pack_kernel(N, D)(x)

    agg = _make_agg_kernel(D, E)(x_pk, ptr_pad, idx_pad)

    b2 = b_l.reshape(1, H)

    BN = 1000
    out = pl.pallas_call(
        _mm_kernel,
        grid=(N // BN,),
        in_specs=[
            pl.BlockSpec((BN, D), lambda i: (i, 0)),
            pl.BlockSpec((BN, D), lambda i: (i, 0)),
            pl.BlockSpec((D, H), lambda i: (0, 0)),
            pl.BlockSpec((D, H), lambda i: (0, 0)),
            pl.BlockSpec((1, H), lambda i: (0, 0)),
        ],
        out_specs=pl.BlockSpec((BN, H), lambda i: (i, 0)),
        out_shape=jax.ShapeDtypeStruct((N, H), jnp.float32),
    )(agg, x, W_l.astype(jnp.bfloat16), W_r.astype(jnp.bfloat16), b2)

    return out
